# Initial kernel scaffold; baseline (speedup 1.0000x reference)
import functools

import jax
import jax.numpy as jnp
from jax import lax
from jax.experimental import pallas as pl
from jax.experimental.pallas import tpu as pltpu
from jax.experimental.pallas import tpu_sc as plsc

N_NODES = 10000
N_EDGES = 160000
D_FEAT = 256
HIDDEN = 16

NUM_WORKERS = 32
EDGES_PER_WORKER = 5008
E_PAD = NUM_WORKERS * EDGES_PER_WORKER
T_PAD = 10240
LANES = 16


def _node_mlp_body(x_ref, wl_ref, bl_ref, wc_ref, o_ref):
    h = jnp.dot(x_ref[...], wl_ref[...], preferred_element_type=jnp.float32)
    h = jnp.maximum(h + bl_ref[...], 0.0)
    o_ref[...] = jnp.dot(h, wc_ref[...], preferred_element_type=jnp.float32)


def _node_mlp(x, w_low, b_low, w_cls):
    return pl.pallas_call(
        _node_mlp_body,
        grid=(5,),
        in_specs=[
            pl.BlockSpec((2000, D_FEAT), lambda i: (i, 0)),
            pl.BlockSpec((D_FEAT, HIDDEN), lambda i: (0, 0)),
            pl.BlockSpec((1, HIDDEN), lambda i: (0, 0)),
            pl.BlockSpec((HIDDEN, 1), lambda i: (0, 0)),
        ],
        out_specs=pl.BlockSpec((2000, 1), lambda i: (i, 0)),
        out_shape=jax.ShapeDtypeStruct((N_NODES, 1), jnp.float32),
    )(x, w_low, b_low.reshape(1, HIDDEN), w_cls)


_SC_MESH = plsc.VectorSubcoreMesh(core_axis_name="c", subcore_axis_name="s")


@functools.partial(
    pl.kernel,
    out_type=jax.ShapeDtypeStruct((NUM_WORKERS, T_PAD), jnp.float32),
    mesh=_SC_MESH,
    scratch_types=[
        pltpu.VMEM((N_NODES,), jnp.float32),
        pltpu.VMEM((EDGES_PER_WORKER,), jnp.int32),
        pltpu.VMEM((EDGES_PER_WORKER,), jnp.int32),
        pltpu.VMEM((T_PAD,), jnp.float32),
    ],
)
def _edge_scatter(s_hbm, src_hbm, dst_hbm, zeros_hbm, out_hbm,
                  s_v, src_v, dst_v, t_v):
    wid = lax.axis_index("s") * 2 + lax.axis_index("c")
    base = wid * EDGES_PER_WORKER
    pltpu.sync_copy(s_hbm, s_v)
    pltpu.sync_copy(src_hbm.at[pl.ds(base, EDGES_PER_WORKER)], src_v)
    pltpu.sync_copy(dst_hbm.at[pl.ds(base, EDGES_PER_WORKER)], dst_v)
    pltpu.sync_copy(zeros_hbm, t_v)

    def body(i, carry):
        si = src_v[pl.ds(i * LANES, LANES)]
        di = dst_v[pl.ds(i * LANES, LANES)]
        vals = plsc.load_gather(s_v, [si])
        plsc.addupdate_scatter(t_v, [di], vals)
        return carry

    lax.fori_loop(0, EDGES_PER_WORKER // LANES, body, 0)
    pltpu.sync_copy(t_v, out_hbm.at[wid])


def _reduce_sigmoid_body(p_ref, bc_ref, o_ref):
    t = jnp.sum(p_ref[...], axis=0, keepdims=True) + bc_ref[...]
    o_ref[...] = jax.nn.sigmoid(t)


def _reduce_sigmoid(partials, b_cls):
    return pl.pallas_call(
        _reduce_sigmoid_body,
        out_shape=jax.ShapeDtypeStruct((1, T_PAD), jnp.float32),
    )(partials, b_cls.reshape(1, 1))


def kernel(x, edge_index, W_low, b_low, W_cls, b_cls):
    src = edge_index[0].astype(jnp.int32)
    dst = edge_index[1].astype(jnp.int32)
    pad = E_PAD - N_EDGES
    src_p = jnp.concatenate([src, jnp.zeros((pad,), jnp.int32)])
    dst_p = jnp.concatenate([dst, jnp.full((pad,), N_NODES, jnp.int32)])
    zeros = jnp.zeros((T_PAD,), jnp.float32)

    s2d = _node_mlp(x, W_low, b_low, W_cls)
    partials = _edge_scatter(s2d.reshape(N_NODES), src_p, dst_p, zeros)
    outr = _reduce_sigmoid(partials, b_cls)
    return outr[0, :N_NODES].reshape(N_NODES, 1)


# speedup vs baseline: 17.0853x; 17.0853x over previous
import functools

import jax
import jax.numpy as jnp
from jax import lax
from jax.experimental import pallas as pl
from jax.experimental.pallas import tpu as pltpu
from jax.experimental.pallas import tpu_sc as plsc

N_NODES = 10000
N_EDGES = 160000
D_FEAT = 256
HIDDEN = 16

NUM_WORKERS = 32
EDGES_PER_WORKER = 5008
E_PAD = NUM_WORKERS * EDGES_PER_WORKER
T_PAD = 10240
LANES = 16


def _node_mlp_body(x_ref, wl_ref, bl_ref, wc_ref, o_ref):
    h = jnp.dot(x_ref[...], wl_ref[...], preferred_element_type=jnp.float32)
    h = jnp.maximum(h + bl_ref[...], 0.0)
    o_ref[...] = jnp.dot(h, wc_ref[...], preferred_element_type=jnp.float32)


def _node_mlp(x, w_low, b_low, w_cls):
    return pl.pallas_call(
        _node_mlp_body,
        grid=(5,),
        in_specs=[
            pl.BlockSpec((2000, D_FEAT), lambda i: (i, 0)),
            pl.BlockSpec((D_FEAT, HIDDEN), lambda i: (0, 0)),
            pl.BlockSpec((1, HIDDEN), lambda i: (0, 0)),
            pl.BlockSpec((HIDDEN, 1), lambda i: (0, 0)),
        ],
        out_specs=pl.BlockSpec((2000, 1), lambda i: (i, 0)),
        out_shape=jax.ShapeDtypeStruct((N_NODES, 1), jnp.float32),
    )(x, w_low, b_low.reshape(1, HIDDEN), w_cls)


_SC_MESH = plsc.VectorSubcoreMesh(core_axis_name="c", subcore_axis_name="s")


@functools.partial(
    pl.kernel,
    out_type=jax.ShapeDtypeStruct((NUM_WORKERS, T_PAD), jnp.float32),
    mesh=_SC_MESH,
    compiler_params=pltpu.CompilerParams(needs_layout_passes=False),
    scratch_types=[
        pltpu.VMEM((N_NODES,), jnp.float32),
        pltpu.VMEM((EDGES_PER_WORKER,), jnp.int32),
        pltpu.VMEM((EDGES_PER_WORKER,), jnp.int32),
        pltpu.VMEM((T_PAD,), jnp.float32),
    ],
)
def _edge_scatter(s_hbm, src_hbm, dst_hbm, zeros_hbm, out_hbm,
                  s_v, src_v, dst_v, t_v):
    wid = lax.axis_index("s") * 2 + lax.axis_index("c")
    base = wid * EDGES_PER_WORKER
    pltpu.sync_copy(s_hbm, s_v)
    pltpu.sync_copy(src_hbm.at[pl.ds(base, EDGES_PER_WORKER)], src_v)
    pltpu.sync_copy(dst_hbm.at[pl.ds(base, EDGES_PER_WORKER)], dst_v)
    pltpu.sync_copy(zeros_hbm, t_v)

    def body(i, carry):
        si = src_v[pl.ds(i * LANES, LANES)]
        di = dst_v[pl.ds(i * LANES, LANES)]
        vals = plsc.load_gather(s_v, [si])
        plsc.addupdate_scatter(t_v, [di], vals)
        return carry

    lax.fori_loop(0, EDGES_PER_WORKER // LANES, body, 0)
    pltpu.sync_copy(t_v, out_hbm.at[wid])


def _reduce_sigmoid_body(p_ref, bc_ref, o_ref):
    t = jnp.sum(p_ref[...], axis=0, keepdims=True) + bc_ref[...]
    o_ref[...] = jax.nn.sigmoid(t)


def _reduce_sigmoid(partials, b_cls):
    return pl.pallas_call(
        _reduce_sigmoid_body,
        out_shape=jax.ShapeDtypeStruct((1, T_PAD), jnp.float32),
    )(partials, b_cls.reshape(1, 1))


def kernel(x, edge_index, W_low, b_low, W_cls, b_cls):
    src = edge_index[0].astype(jnp.int32)
    dst = edge_index[1].astype(jnp.int32)
    pad = E_PAD - N_EDGES
    src_p = jnp.concatenate([src, jnp.zeros((pad,), jnp.int32)])
    dst_p = jnp.concatenate([dst, jnp.full((pad,), N_NODES, jnp.int32)])
    zeros = jnp.zeros((T_PAD,), jnp.float32)

    s2d = _node_mlp(x, W_low, b_low, W_cls)
    partials = _edge_scatter(s2d.reshape(N_NODES), src_p, dst_p, zeros)
    outr = _reduce_sigmoid(partials, b_cls)
    return outr[0, :N_NODES].reshape(N_NODES, 1)


# speedup vs baseline: 21.1590x; 1.2384x over previous
import functools

import jax
import jax.numpy as jnp
from jax import lax
from jax.experimental import pallas as pl
from jax.experimental.pallas import tpu as pltpu
from jax.experimental.pallas import tpu_sc as plsc

N_NODES = 10000
N_EDGES = 160000
D_FEAT = 256
HIDDEN = 16

NUM_WORKERS = 32
EDGES_PER_WORKER = N_EDGES // NUM_WORKERS
FULL_ITERS = EDGES_PER_WORKER // 16
TAIL_OFF = EDGES_PER_WORKER - 16
T_PAD = 10240
LANES = 16
ROWS_PER_BLOCK = 1024


def _node_mlp_body(x_ref, wl_ref, bl_ref, wc_ref, o_ref):
    h = jnp.dot(x_ref[...], wl_ref[...], preferred_element_type=jnp.float32)
    g = jnp.maximum(h + bl_ref[...], 0.0) * wc_ref[...]
    o_ref[...] = jnp.sum(g, axis=1)


def _node_mlp(x, w_low, b_low, w_cls):
    return pl.pallas_call(
        _node_mlp_body,
        grid=(T_PAD // ROWS_PER_BLOCK,),
        in_specs=[
            pl.BlockSpec((ROWS_PER_BLOCK, D_FEAT), lambda i: (i, 0)),
            pl.BlockSpec((D_FEAT, HIDDEN), lambda i: (0, 0)),
            pl.BlockSpec((1, HIDDEN), lambda i: (0, 0)),
            pl.BlockSpec((1, HIDDEN), lambda i: (0, 0)),
        ],
        out_specs=pl.BlockSpec((ROWS_PER_BLOCK,), lambda i: (i,)),
        out_shape=jax.ShapeDtypeStruct((T_PAD,), jnp.float32),
    )(x, w_low, b_low.reshape(1, HIDDEN), w_cls.reshape(1, HIDDEN))


_SC_MESH = plsc.VectorSubcoreMesh(core_axis_name="c", subcore_axis_name="s")


@functools.partial(
    pl.kernel,
    out_type=jax.ShapeDtypeStruct((NUM_WORKERS, T_PAD), jnp.float32),
    mesh=_SC_MESH,
    compiler_params=pltpu.CompilerParams(needs_layout_passes=False),
    scratch_types=[
        pltpu.VMEM((T_PAD,), jnp.float32),
        pltpu.VMEM((EDGES_PER_WORKER,), jnp.int32),
        pltpu.VMEM((EDGES_PER_WORKER,), jnp.int32),
        pltpu.VMEM((T_PAD,), jnp.float32),
    ],
)
def _edge_scatter(s_hbm, edges_hbm, zeros_hbm, out_hbm,
                  s_v, src_v, dst_v, t_v):
    wid = lax.axis_index("s") * 2 + lax.axis_index("c")
    base = wid * EDGES_PER_WORKER
    pltpu.sync_copy(s_hbm, s_v)
    pltpu.sync_copy(edges_hbm.at[pl.ds(base, EDGES_PER_WORKER)], src_v)
    pltpu.sync_copy(edges_hbm.at[pl.ds(N_EDGES + base, EDGES_PER_WORKER)],
                    dst_v)
    pltpu.sync_copy(zeros_hbm, t_v)

    def body(i, carry):
        si = src_v[pl.ds(i * LANES, LANES)]
        di = dst_v[pl.ds(i * LANES, LANES)]
        vals = plsc.load_gather(s_v, [si])
        plsc.addupdate_scatter(t_v, [di], vals)
        return carry

    lax.fori_loop(0, FULL_ITERS, body, 0, unroll=8)

    mask = lax.iota(jnp.int32, LANES) >= (LANES - 8)
    si = src_v[pl.ds(TAIL_OFF, LANES)]
    di = dst_v[pl.ds(TAIL_OFF, LANES)]
    vals = plsc.load_gather(s_v, [si], mask=mask)
    plsc.addupdate_scatter(t_v, [di], vals, mask=mask)

    pltpu.sync_copy(t_v, out_hbm.at[wid])


def _reduce_sigmoid_body(p_ref, bc_ref, o_ref):
    t = jnp.sum(p_ref[...], axis=0, keepdims=True) + bc_ref[...]
    o_ref[...] = jax.nn.sigmoid(t)


def _reduce_sigmoid(partials, b_cls):
    return pl.pallas_call(
        _reduce_sigmoid_body,
        out_shape=jax.ShapeDtypeStruct((1, T_PAD), jnp.float32),
    )(partials, b_cls.reshape(1, 1))


def kernel(x, edge_index, W_low, b_low, W_cls, b_cls):
    edges_flat = edge_index.astype(jnp.int32).reshape(2 * N_EDGES)
    zeros = jnp.zeros((T_PAD,), jnp.float32)

    s = _node_mlp(x, W_low, b_low, W_cls)
    partials = _edge_scatter(s, edges_flat, zeros)
    outr = _reduce_sigmoid(partials, b_cls)
    return outr[0, :N_NODES].reshape(N_NODES, 1)
